# baseline (speedup 1.0000x reference)
import functools

import jax
import jax.numpy as jnp
from jax.experimental import pallas as pl
from jax.experimental.pallas import tpu as pltpu


def _lstm_body(xs_ref, wx_ref, wh_ref, b_ref, out_ref, cf_ref, hf_ref,
               xp_ref, c_ref, h_ref, *, T, mem, batch):
    k = pl.program_id(0)
    G = pl.num_programs(0)

    @pl.when(k == 0)
    def _():
        c_ref[:] = jnp.zeros_like(c_ref)
        h_ref[:] = jnp.zeros_like(h_ref)

    x2 = xs_ref[:].reshape(T * batch, -1)
    xp_ref[:] = jnp.dot(x2, wx_ref[:], preferred_element_type=jnp.float32) + b_ref[:]

    def step(t, carry):
        c, h = carry
        z = xp_ref[pl.ds(t * batch, batch), :] + jnp.dot(
            h.astype(jnp.bfloat16), wh_ref[:], preferred_element_type=jnp.float32)
        i = jax.nn.sigmoid(z[:, :mem])
        o = jax.nn.sigmoid(z[:, mem:2 * mem])
        u = jax.nn.sigmoid(z[:, 2 * mem:3 * mem])
        f = jax.nn.sigmoid(z[:, 3 * mem:])
        c = i * u + f * c
        h = o * jnp.tanh(c)
        out_ref[t] = h
        return (c, h)

    c, h = jax.lax.fori_loop(0, T, step, (c_ref[:], h_ref[:]))
    c_ref[:] = c
    h_ref[:] = h

    @pl.when(k == G - 1)
    def _():
        cf_ref[:] = c
        hf_ref[:] = h


def kernel(trees, inputs, ioux_w, ioux_b, iouh_w, iouh_b, fx_w, fx_b, fh_w, fh_b):
    del trees
    b, n, in_dim = inputs.shape
    mem = fx_b.shape[0]
    wx = jnp.concatenate([ioux_w, fx_w], axis=0).T
    wh = jnp.concatenate([iouh_w, fh_w], axis=0).T.astype(jnp.bfloat16)
    bias = jnp.concatenate([ioux_b + iouh_b, fx_b + fh_b])[None, :]
    xs = inputs[:, ::-1, :].transpose(1, 0, 2)

    T = 64
    G = n // T
    body = functools.partial(_lstm_body, T=T, mem=mem, batch=b)
    hs, c_fin, h_fin = pl.pallas_call(
        body,
        grid=(G,),
        in_specs=[
            pl.BlockSpec((T, b, in_dim), lambda k: (k, 0, 0)),
            pl.BlockSpec((in_dim, 4 * mem), lambda k: (0, 0)),
            pl.BlockSpec((mem, 4 * mem), lambda k: (0, 0)),
            pl.BlockSpec((1, 4 * mem), lambda k: (0, 0)),
        ],
        out_specs=[
            pl.BlockSpec((T, b, mem), lambda k: (k, 0, 0)),
            pl.BlockSpec((b, mem), lambda k: (0, 0)),
            pl.BlockSpec((b, mem), lambda k: (0, 0)),
        ],
        out_shape=[
            jax.ShapeDtypeStruct((n, b, mem), jnp.float32),
            jax.ShapeDtypeStruct((b, mem), jnp.float32),
            jax.ShapeDtypeStruct((b, mem), jnp.float32),
        ],
        scratch_shapes=[
            pltpu.VMEM((T * b, 4 * mem), jnp.float32),
            pltpu.VMEM((b, mem), jnp.float32),
            pltpu.VMEM((b, mem), jnp.float32),
        ],
    )(xs, wx, wh, bias)
    o_states = hs[::-1].transpose(1, 0, 2)
    return (o_states, c_fin, h_fin)


# speedup vs baseline: 1.0186x; 1.0186x over previous
import functools

import jax
import jax.numpy as jnp
from jax.experimental import pallas as pl
from jax.experimental.pallas import tpu as pltpu


def _lstm_body(xs_ref, wx_ref, wh_ref, b_ref, out_ref, cf_ref, hf_ref,
               xp_ref, c_ref, h_ref, *, T, mem, batch):
    k = pl.program_id(0)
    G = pl.num_programs(0)

    @pl.when(k == 0)
    def _():
        c_ref[:] = jnp.zeros_like(c_ref)
        h_ref[:] = jnp.zeros_like(h_ref)

    x2 = xs_ref[:].reshape(T * batch, -1)
    xp_ref[:] = jnp.dot(x2, wx_ref[:], preferred_element_type=jnp.float32) + b_ref[:]

    def step(t, carry):
        c, h = carry
        z = xp_ref[pl.ds(t * batch, batch), :] + jnp.dot(
            h.astype(jnp.bfloat16), wh_ref[:], preferred_element_type=jnp.float32)
        ti = jnp.tanh(z[:, :mem])
        to = jnp.tanh(z[:, mem:2 * mem])
        tu = jnp.tanh(z[:, 2 * mem:3 * mem])
        tf = jnp.tanh(z[:, 3 * mem:])
        c = 0.25 * (ti * tu + ti + tu + 1.0) + 0.5 * (tf * c + c)
        tc = jnp.tanh(c)
        h = 0.5 * (to * tc + tc)
        out_ref[t] = h
        return (c, h)

    c, h = jax.lax.fori_loop(0, T, step, (c_ref[:], h_ref[:]))
    c_ref[:] = c
    h_ref[:] = h

    @pl.when(k == G - 1)
    def _():
        cf_ref[:] = c
        hf_ref[:] = h


def kernel(trees, inputs, ioux_w, ioux_b, iouh_w, iouh_b, fx_w, fx_b, fh_w, fh_b):
    del trees
    b, n, in_dim = inputs.shape
    mem = fx_b.shape[0]
    wx = 0.5 * jnp.concatenate([ioux_w, fx_w], axis=0).T
    wh = (0.5 * jnp.concatenate([iouh_w, fh_w], axis=0).T).astype(jnp.bfloat16)
    bias = 0.5 * jnp.concatenate([ioux_b + iouh_b, fx_b + fh_b])[None, :]
    xs = inputs[:, ::-1, :].transpose(1, 0, 2)

    T = 64
    G = n // T
    body = functools.partial(_lstm_body, T=T, mem=mem, batch=b)
    hs, c_fin, h_fin = pl.pallas_call(
        body,
        grid=(G,),
        in_specs=[
            pl.BlockSpec((T, b, in_dim), lambda k: (k, 0, 0)),
            pl.BlockSpec((in_dim, 4 * mem), lambda k: (0, 0)),
            pl.BlockSpec((mem, 4 * mem), lambda k: (0, 0)),
            pl.BlockSpec((1, 4 * mem), lambda k: (0, 0)),
        ],
        out_specs=[
            pl.BlockSpec((T, b, mem), lambda k: (k, 0, 0)),
            pl.BlockSpec((b, mem), lambda k: (0, 0)),
            pl.BlockSpec((b, mem), lambda k: (0, 0)),
        ],
        out_shape=[
            jax.ShapeDtypeStruct((n, b, mem), jnp.float32),
            jax.ShapeDtypeStruct((b, mem), jnp.float32),
            jax.ShapeDtypeStruct((b, mem), jnp.float32),
        ],
        scratch_shapes=[
            pltpu.VMEM((T * b, 4 * mem), jnp.float32),
            pltpu.VMEM((b, mem), jnp.float32),
            pltpu.VMEM((b, mem), jnp.float32),
        ],
    )(xs, wx, wh, bias)
    o_states = hs[::-1].transpose(1, 0, 2)
    return (o_states, c_fin, h_fin)
